# baseline (speedup 1.0000x reference)
import jax
import jax.numpy as jnp
from jax import lax
from jax.experimental import pallas as pl
from jax.experimental.pallas import tpu as pltpu
from jax.experimental.pallas import tpu_sc as plsc

_N = 100000
_E = 1600000
_D_IN = 128
_H = 32
_HH = 16
_EC = 128
_NC = 2
_NS = 16
_NPT = 6256
_NPAD = _NPT * _NS
_R = _E // _EC
_NBLK = _R // 8

_mesh = plsc.VectorSubcoreMesh(core_axis_name="c", subcore_axis_name="s")
_sc_params = pltpu.CompilerParams(use_tc_tiling_on_sc=False)


def _deg_body(dst2, basis, zeros_tab, out0, out1,
              deg_sp, bbuf, dbuf, ssem):
    c = lax.axis_index("c")
    s = lax.axis_index("s")
    rows0 = pl.multiple_of(s * _NPT, 8)

    pltpu.sync_copy(zeros_tab.at[pl.ds(rows0, _NPT)],
                    deg_sp.at[pl.ds(rows0, _NPT)])
    pltpu.sync_copy(basis, bbuf)
    plsc.subcore_barrier()

    w = s * _NC + c
    start = 48 * w + jnp.minimum(w, 26)

    def do_block(r0, n):
        pltpu.sync_copy(dst2.at[pl.ds(r0, n)], dbuf.at[pl.ds(0, n)])
        adds = [pltpu.async_copy(bbuf, deg_sp.at[dbuf.at[j]], ssem, add=True)
                for j in range(n)]
        for a in adds:
            a.wait()

    @pl.loop(0, 48)
    def _blk(k):
        do_block(pl.multiple_of((start + k) * 8, 8), 8)

    @pl.when(w < 26)
    def _():
        do_block(pl.multiple_of((start + 48) * 8, 8), 8)

    @pl.when(w == 31)
    def _():
        do_block(_NBLK * 8, 4)

    plsc.subcore_barrier()

    @pl.when(c == 0)
    def _():
        pltpu.sync_copy(deg_sp.at[pl.ds(rows0, _NPT)],
                        out0.at[pl.ds(rows0, _NPT)])

    @pl.when(c == 1)
    def _():
        pltpu.sync_copy(deg_sp.at[pl.ds(rows0, _NPT)],
                        out1.at[pl.ds(rows0, _NPT)])


_deg_call = pl.kernel(
    _deg_body,
    out_type=(jax.ShapeDtypeStruct((_NPAD, _HH), jnp.float32),
              jax.ShapeDtypeStruct((_NPAD, _HH), jnp.float32)),
    mesh=_mesh,
    scratch_types=[
        pltpu.VMEM_SHARED((_NPAD, _HH), jnp.float32),
        pltpu.VMEM((_EC, _HH), jnp.float32),
        pltpu.VMEM((8, _EC), jnp.int32),
        pltpu.SemaphoreType.DMA,
    ],
    compiler_params=_sc_params,
)


def _agg_body(zlo, zhi, src2, dst2, outlo, outhi,
              agg_sp, sbA, dbA, sbB, dbB, msg, isem, gsem, ssem):
    c = lax.axis_index("c")
    s = lax.axis_index("s")
    rows0 = pl.multiple_of(s * _NPT, 8)
    start = 97 * s + jnp.minimum(s, 10)

    def idx_fire(blk, sb, db):
        r0 = pl.multiple_of((start + blk) * 8, 8)
        pltpu.async_copy(src2.at[pl.ds(r0, 8)], sb, isem)
        pltpu.async_copy(dst2.at[pl.ds(r0, 8)], db, isem)

    def idx_drain(sb, db):
        pltpu.make_async_copy(src2.at[pl.ds(0, 8)], sb, isem).wait()
        pltpu.make_async_copy(dst2.at[pl.ds(0, 8)], db, isem).wait()

    def run(z_ref, out_ref):
        def do_rows(sb, db, n):
            gs = [pltpu.async_copy(z_ref.at[sb.at[j]], msg.at[j], gsem)
                  for j in range(n)]
            for g in gs:
                g.wait()
            adds = [pltpu.async_copy(msg.at[j], agg_sp.at[db.at[j]], ssem,
                                     add=True)
                    for j in range(n)]
            for a in adds:
                a.wait()

        pltpu.sync_copy(z_ref.at[pl.ds(rows0, _NPT)],
                        agg_sp.at[pl.ds(rows0, _NPT)])
        plsc.subcore_barrier()

        @pl.loop(0, 97)
        def _blk(k):
            r0 = pl.multiple_of((start + k) * 8, 8)
            pltpu.sync_copy(src2.at[pl.ds(r0, 8)], sbA)
            pltpu.sync_copy(dst2.at[pl.ds(r0, 8)], dbA)
            do_rows(sbA, dbA, 8)

        @pl.when(s < 10)
        def _():
            r0 = pl.multiple_of((start + 97) * 8, 8)
            pltpu.sync_copy(src2.at[pl.ds(r0, 8)], sbB)
            pltpu.sync_copy(dst2.at[pl.ds(r0, 8)], dbB)
            do_rows(sbB, dbB, 8)

        @pl.when(s == 15)
        def _():
            pltpu.sync_copy(src2.at[pl.ds(_NBLK * 8, 4)],
                            sbB.at[pl.ds(0, 4)])
            pltpu.sync_copy(dst2.at[pl.ds(_NBLK * 8, 4)],
                            dbB.at[pl.ds(0, 4)])
            do_rows(sbB, dbB, 4)

        plsc.subcore_barrier()
        pltpu.sync_copy(agg_sp.at[pl.ds(rows0, _NPT)],
                        out_ref.at[pl.ds(rows0, _NPT)])

    @pl.when(c == 0)
    def _():
        run(zlo, outlo)

    @pl.when(c == 1)
    def _():
        run(zhi, outhi)


_agg_call = pl.kernel(
    _agg_body,
    out_type=(jax.ShapeDtypeStruct((_NPAD, _HH), jnp.float32),
              jax.ShapeDtypeStruct((_NPAD, _HH), jnp.float32)),
    mesh=_mesh,
    scratch_types=[
        pltpu.VMEM_SHARED((_NPAD, _HH), jnp.float32),
        pltpu.VMEM((8, _EC), jnp.int32),
        pltpu.VMEM((8, _EC), jnp.int32),
        pltpu.VMEM((8, _EC), jnp.int32),
        pltpu.VMEM((8, _EC), jnp.int32),
        pltpu.VMEM((8, _EC, _HH), jnp.float32),
        pltpu.SemaphoreType.DMA,
        pltpu.SemaphoreType.DMA,
        pltpu.SemaphoreType.DMA,
    ],
    compiler_params=_sc_params,
)


_FR = _NPAD * _HH // 128
_XR = _N * _D_IN // 1024
_BN = 2048
_BF = _BN * _HH // 128
_GPAD = (_FR + _BF - 1) // _BF
_HP = lax.Precision.HIGHEST


def _tc_first_body(x_ref, klo_ref, khi_ref, d0_ref, d1_ref, s_ref,
                   zlo_ref, zhi_ref, dinv_ref):
    dband = jnp.dot(d0_ref[...] + d1_ref[...], s_ref[...],
                    preferred_element_type=jnp.float32, precision=_HP)
    dinv = lax.rsqrt(dband + 1.0)
    xb = x_ref[...]
    zlo_ref[...] = jnp.dot(xb, klo_ref[...],
                           preferred_element_type=jnp.float32,
                           precision=_HP) * dinv
    zhi_ref[...] = jnp.dot(xb, khi_ref[...],
                           preferred_element_type=jnp.float32,
                           precision=_HP) * dinv
    dinv_ref[...] = dinv


_tc_first = pl.pallas_call(
    _tc_first_body,
    grid=(_GPAD,),
    in_specs=[
        pl.BlockSpec((_BF, 1024), lambda i: (i, 0)),
        pl.BlockSpec((1024, 128), lambda i: (0, 0)),
        pl.BlockSpec((1024, 128), lambda i: (0, 0)),
        pl.BlockSpec((_BF, 128), lambda i: (i, 0)),
        pl.BlockSpec((_BF, 128), lambda i: (i, 0)),
        pl.BlockSpec((128, 128), lambda i: (0, 0)),
    ],
    out_specs=(
        pl.BlockSpec((_BF, 128), lambda i: (i, 0)),
        pl.BlockSpec((_BF, 128), lambda i: (i, 0)),
        pl.BlockSpec((_BF, 128), lambda i: (i, 0)),
    ),
    out_shape=(
        jax.ShapeDtypeStruct((_FR, 128), jnp.float32),
        jax.ShapeDtypeStruct((_FR, 128), jnp.float32),
        jax.ShapeDtypeStruct((_FR, 128), jnp.float32),
    ),
)


def _tc_mid_body(alo_ref, ahi_ref, kll_ref, khl_ref, klh_ref, khh_ref,
                 blo_ref, bhi_ref, dinv_ref, zlo_ref, zhi_ref):
    dinv = dinv_ref[...]
    hlo = jnp.maximum(alo_ref[...] * dinv + blo_ref[...], 0.0)
    hhi = jnp.maximum(ahi_ref[...] * dinv + bhi_ref[...], 0.0)
    zlo = (jnp.dot(hlo, kll_ref[...], preferred_element_type=jnp.float32,
                   precision=_HP)
           + jnp.dot(hhi, khl_ref[...], preferred_element_type=jnp.float32,
                     precision=_HP))
    zhi = (jnp.dot(hlo, klh_ref[...], preferred_element_type=jnp.float32,
                   precision=_HP)
           + jnp.dot(hhi, khh_ref[...], preferred_element_type=jnp.float32,
                     precision=_HP))
    zlo_ref[...] = zlo * dinv
    zhi_ref[...] = zhi * dinv


_tc_mid = pl.pallas_call(
    _tc_mid_body,
    grid=(_GPAD,),
    in_specs=[
        pl.BlockSpec((_BF, 128), lambda i: (i, 0)),
        pl.BlockSpec((_BF, 128), lambda i: (i, 0)),
        pl.BlockSpec((128, 128), lambda i: (0, 0)),
        pl.BlockSpec((128, 128), lambda i: (0, 0)),
        pl.BlockSpec((128, 128), lambda i: (0, 0)),
        pl.BlockSpec((128, 128), lambda i: (0, 0)),
        pl.BlockSpec((1, 128), lambda i: (0, 0)),
        pl.BlockSpec((1, 128), lambda i: (0, 0)),
        pl.BlockSpec((_BF, 128), lambda i: (i, 0)),
    ],
    out_specs=(
        pl.BlockSpec((_BF, 128), lambda i: (i, 0)),
        pl.BlockSpec((_BF, 128), lambda i: (i, 0)),
    ),
    out_shape=(
        jax.ShapeDtypeStruct((_FR, 128), jnp.float32),
        jax.ShapeDtypeStruct((_FR, 128), jnp.float32),
    ),
)


def _tc_last_body(alo_ref, ahi_ref, blo_ref, bhi_ref, flo_ref, fhi_ref,
                  ssum_ref, fcb_ref, dinv_ref, y_ref):
    dinv = dinv_ref[...]
    hlo = jnp.maximum(alo_ref[...] * dinv + blo_ref[...], 0.0)
    hhi = jnp.maximum(ahi_ref[...] * dinv + bhi_ref[...], 0.0)
    t = hlo * flo_ref[...] + hhi * fhi_ref[...]
    y_ref[...] = jnp.dot(t, ssum_ref[...], preferred_element_type=jnp.float32,
                         precision=_HP) + fcb_ref[...]


_tc_last = pl.pallas_call(
    _tc_last_body,
    grid=(_GPAD,),
    in_specs=[
        pl.BlockSpec((_BF, 128), lambda i: (i, 0)),
        pl.BlockSpec((_BF, 128), lambda i: (i, 0)),
        pl.BlockSpec((1, 128), lambda i: (0, 0)),
        pl.BlockSpec((1, 128), lambda i: (0, 0)),
        pl.BlockSpec((1, 128), lambda i: (0, 0)),
        pl.BlockSpec((1, 128), lambda i: (0, 0)),
        pl.BlockSpec((128, 8), lambda i: (0, 0)),
        pl.BlockSpec((1, 8), lambda i: (0, 0)),
        pl.BlockSpec((_BF, 128), lambda i: (i, 0)),
    ],
    out_specs=pl.BlockSpec((_BF, 8), lambda i: (i, 0)),
    out_shape=jax.ShapeDtypeStruct((_FR, 8), jnp.float32),
)


def kernel(x, edge_index, W0, b0, W1, b1, W2, b2, W3, b3, fc_W, fc_b):
    src2 = edge_index[0].reshape(_R, _EC)
    dst2 = edge_index[1].reshape(_R, _EC)
    basis = jnp.zeros((_EC, _HH), jnp.float32).at[:, 0].set(1.0)
    zeros_tab = jnp.zeros((_NPAD, _HH), jnp.float32)

    eye8 = jnp.eye(8, dtype=jnp.float32)
    s_spread = jnp.kron(eye8, jnp.zeros((16, 16), jnp.float32)
                        .at[0, :].set(1.0))
    s_sum = jnp.kron(eye8, jnp.ones((16, 1), jnp.float32))

    def flat(t):
        return t.reshape(_FR, 128)

    def unflat(t):
        return t.reshape(_NPAD, _HH)

    d0, d1 = _deg_call(dst2, basis, zeros_tab)
    zlo_f, zhi_f, dinv_f = _tc_first(
        x.reshape(_XR, 1024),
        jnp.kron(eye8, W0[:, :_HH]), jnp.kron(eye8, W0[:, _HH:]),
        flat(d0), flat(d1), s_spread)
    for (W, b) in ((W1, b0), (W2, b1), (W3, b2)):
        alo, ahi = _agg_call(unflat(zlo_f), unflat(zhi_f), src2, dst2)
        zlo_f, zhi_f = _tc_mid(
            flat(alo), flat(ahi),
            jnp.kron(eye8, W[:_HH, :_HH]), jnp.kron(eye8, W[_HH:, :_HH]),
            jnp.kron(eye8, W[:_HH, _HH:]), jnp.kron(eye8, W[_HH:, _HH:]),
            jnp.tile(b[:_HH], 8).reshape(1, 128),
            jnp.tile(b[_HH:], 8).reshape(1, 128),
            dinv_f)
    alo, ahi = _agg_call(unflat(zlo_f), unflat(zhi_f), src2, dst2)
    y8 = _tc_last(
        flat(alo), flat(ahi),
        jnp.tile(b3[:_HH], 8).reshape(1, 128),
        jnp.tile(b3[_HH:], 8).reshape(1, 128),
        jnp.tile(fc_W[:_HH, 0], 8).reshape(1, 128),
        jnp.tile(fc_W[_HH:, 0], 8).reshape(1, 128),
        s_sum, jnp.tile(fc_b, 8).reshape(1, 8), dinv_f)
    return y8.reshape(_NPAD)[:_N]


# speedup vs baseline: 1.1756x; 1.1756x over previous
import jax
import jax.numpy as jnp
from jax import lax
from jax.experimental import pallas as pl
from jax.experimental.pallas import tpu as pltpu
from jax.experimental.pallas import tpu_sc as plsc

_N = 100000
_E = 1600000
_D_IN = 128
_H = 32
_HH = 16
_EC = 128
_NC = 2
_NS = 16
_NPT = 6256
_NPAD = _NPT * _NS
_R = _E // _EC
_NBLK = _R // 8

_mesh = plsc.VectorSubcoreMesh(core_axis_name="c", subcore_axis_name="s")
_sc_params = pltpu.CompilerParams(use_tc_tiling_on_sc=False)


def _deg_body(dst2, basis, zeros_tab, out0, out1,
              deg_sp, bbuf, dbuf, ssem):
    c = lax.axis_index("c")
    s = lax.axis_index("s")
    rows0 = pl.multiple_of(s * _NPT, 8)

    pltpu.sync_copy(zeros_tab.at[pl.ds(rows0, _NPT)],
                    deg_sp.at[pl.ds(rows0, _NPT)])
    pltpu.sync_copy(basis, bbuf)
    plsc.subcore_barrier()

    w = s * _NC + c
    start = 48 * w + jnp.minimum(w, 26)

    def do_block(r0, n):
        pltpu.sync_copy(dst2.at[pl.ds(r0, n)], dbuf.at[pl.ds(0, n)])
        adds = [pltpu.async_copy(bbuf, deg_sp.at[dbuf.at[j]], ssem, add=True)
                for j in range(n)]
        for a in adds:
            a.wait()

    @pl.loop(0, 48)
    def _blk(k):
        do_block(pl.multiple_of((start + k) * 8, 8), 8)

    @pl.when(w < 26)
    def _():
        do_block(pl.multiple_of((start + 48) * 8, 8), 8)

    @pl.when(w == 31)
    def _():
        do_block(_NBLK * 8, 4)

    plsc.subcore_barrier()

    @pl.when(c == 0)
    def _():
        pltpu.sync_copy(deg_sp.at[pl.ds(rows0, _NPT)],
                        out0.at[pl.ds(rows0, _NPT)])

    @pl.when(c == 1)
    def _():
        pltpu.sync_copy(deg_sp.at[pl.ds(rows0, _NPT)],
                        out1.at[pl.ds(rows0, _NPT)])


_deg_call = pl.kernel(
    _deg_body,
    out_type=(jax.ShapeDtypeStruct((_NPAD, _HH), jnp.float32),
              jax.ShapeDtypeStruct((_NPAD, _HH), jnp.float32)),
    mesh=_mesh,
    scratch_types=[
        pltpu.VMEM_SHARED((_NPAD, _HH), jnp.float32),
        pltpu.VMEM((_EC, _HH), jnp.float32),
        pltpu.VMEM((8, _EC), jnp.int32),
        pltpu.SemaphoreType.DMA,
    ],
    compiler_params=_sc_params,
)


def _agg_body(zlo, zhi, src2, dst2, outlo, outhi,
              agg_sp, sbA, dbA, sbB, dbB, msg, isem, gsem, ssem):
    c = lax.axis_index("c")
    s = lax.axis_index("s")
    rows0 = pl.multiple_of(s * _NPT, 8)
    start = 97 * s + jnp.minimum(s, 10)

    def r0_of(k):
        return pl.multiple_of(jnp.minimum(start + k, _NBLK - 1) * 8, 8)

    def idx_fire(r0, sb, db):
        pltpu.async_copy(src2.at[pl.ds(r0, 8)], sb, isem)
        pltpu.async_copy(dst2.at[pl.ds(r0, 8)], db, isem)

    def idx_wait(sb, db):
        pltpu.make_async_copy(src2.at[pl.ds(0, 8)], sb, isem).wait()
        pltpu.make_async_copy(dst2.at[pl.ds(0, 8)], db, isem).wait()

    def run(z_ref, out_ref):
        def do_block8(sb, db):
            gA = [pltpu.async_copy(z_ref.at[sb.at[j]], msg.at[j], gsem)
                  for j in range(4)]
            for g in gA:
                g.wait()
            gB = [pltpu.async_copy(z_ref.at[sb.at[j]], msg.at[j], gsem)
                  for j in range(4, 8)]
            sA = [pltpu.async_copy(msg.at[j], agg_sp.at[db.at[j]], ssem,
                                   add=True)
                  for j in range(4)]
            for g in gB:
                g.wait()
            for a in sA:
                a.wait()
            sB = [pltpu.async_copy(msg.at[j], agg_sp.at[db.at[j]], ssem,
                                   add=True)
                  for j in range(4, 8)]
            for a in sB:
                a.wait()

        def do_rows4(sb, db):
            gs = [pltpu.async_copy(z_ref.at[sb.at[j]], msg.at[j], gsem)
                  for j in range(4)]
            for g in gs:
                g.wait()
            adds = [pltpu.async_copy(msg.at[j], agg_sp.at[db.at[j]], ssem,
                                     add=True)
                    for j in range(4)]
            for a in adds:
                a.wait()

        pltpu.sync_copy(z_ref.at[pl.ds(rows0, _NPT)],
                        agg_sp.at[pl.ds(rows0, _NPT)])
        plsc.subcore_barrier()

        idx_fire(r0_of(0), sbA, dbA)

        @pl.loop(0, 48)
        def _blk2(t):
            kA = 2 * t
            idx_wait(sbA, dbA)
            idx_fire(r0_of(kA + 1), sbB, dbB)
            do_block8(sbA, dbA)
            idx_wait(sbB, dbB)
            idx_fire(r0_of(kA + 2), sbA, dbA)
            do_block8(sbB, dbB)

        idx_wait(sbA, dbA)
        idx_fire(r0_of(97), sbB, dbB)
        do_block8(sbA, dbA)
        idx_wait(sbB, dbB)

        @pl.when(s < 10)
        def _():
            do_block8(sbB, dbB)

        @pl.when(s == 15)
        def _():
            pltpu.sync_copy(src2.at[pl.ds(_NBLK * 8, 4)],
                            sbA.at[pl.ds(0, 4)])
            pltpu.sync_copy(dst2.at[pl.ds(_NBLK * 8, 4)],
                            dbA.at[pl.ds(0, 4)])
            do_rows4(sbA, dbA)

        plsc.subcore_barrier()
        pltpu.sync_copy(agg_sp.at[pl.ds(rows0, _NPT)],
                        out_ref.at[pl.ds(rows0, _NPT)])

    @pl.when(c == 0)
    def _():
        run(zlo, outlo)

    @pl.when(c == 1)
    def _():
        run(zhi, outhi)


_agg_call = pl.kernel(
    _agg_body,
    out_type=(jax.ShapeDtypeStruct((_NPAD, _HH), jnp.float32),
              jax.ShapeDtypeStruct((_NPAD, _HH), jnp.float32)),
    mesh=_mesh,
    scratch_types=[
        pltpu.VMEM_SHARED((_NPAD, _HH), jnp.float32),
        pltpu.VMEM((8, _EC), jnp.int32),
        pltpu.VMEM((8, _EC), jnp.int32),
        pltpu.VMEM((8, _EC), jnp.int32),
        pltpu.VMEM((8, _EC), jnp.int32),
        pltpu.VMEM((8, _EC, _HH), jnp.float32),
        pltpu.SemaphoreType.DMA,
        pltpu.SemaphoreType.DMA,
        pltpu.SemaphoreType.DMA,
    ],
    compiler_params=_sc_params,
)


_FR = _NPAD * _HH // 128
_XR = _N * _D_IN // 1024
_BN = 2048
_BF = _BN * _HH // 128
_GPAD = (_FR + _BF - 1) // _BF
_HP = lax.Precision.HIGHEST


def _tc_first_body(x_ref, klo_ref, khi_ref, d0_ref, d1_ref, s_ref,
                   zlo_ref, zhi_ref, dinv_ref):
    dband = jnp.dot(d0_ref[...] + d1_ref[...], s_ref[...],
                    preferred_element_type=jnp.float32, precision=_HP)
    dinv = lax.rsqrt(dband + 1.0)
    xb = x_ref[...]
    zlo_ref[...] = jnp.dot(xb, klo_ref[...],
                           preferred_element_type=jnp.float32,
                           precision=_HP) * dinv
    zhi_ref[...] = jnp.dot(xb, khi_ref[...],
                           preferred_element_type=jnp.float32,
                           precision=_HP) * dinv
    dinv_ref[...] = dinv


_tc_first = pl.pallas_call(
    _tc_first_body,
    grid=(_GPAD,),
    in_specs=[
        pl.BlockSpec((_BF, 1024), lambda i: (i, 0)),
        pl.BlockSpec((1024, 128), lambda i: (0, 0)),
        pl.BlockSpec((1024, 128), lambda i: (0, 0)),
        pl.BlockSpec((_BF, 128), lambda i: (i, 0)),
        pl.BlockSpec((_BF, 128), lambda i: (i, 0)),
        pl.BlockSpec((128, 128), lambda i: (0, 0)),
    ],
    out_specs=(
        pl.BlockSpec((_BF, 128), lambda i: (i, 0)),
        pl.BlockSpec((_BF, 128), lambda i: (i, 0)),
        pl.BlockSpec((_BF, 128), lambda i: (i, 0)),
    ),
    out_shape=(
        jax.ShapeDtypeStruct((_FR, 128), jnp.float32),
        jax.ShapeDtypeStruct((_FR, 128), jnp.float32),
        jax.ShapeDtypeStruct((_FR, 128), jnp.float32),
    ),
)


def _tc_mid_body(alo_ref, ahi_ref, kll_ref, khl_ref, klh_ref, khh_ref,
                 blo_ref, bhi_ref, dinv_ref, zlo_ref, zhi_ref):
    dinv = dinv_ref[...]
    hlo = jnp.maximum(alo_ref[...] * dinv + blo_ref[...], 0.0)
    hhi = jnp.maximum(ahi_ref[...] * dinv + bhi_ref[...], 0.0)
    zlo = (jnp.dot(hlo, kll_ref[...], preferred_element_type=jnp.float32,
                   precision=_HP)
           + jnp.dot(hhi, khl_ref[...], preferred_element_type=jnp.float32,
                     precision=_HP))
    zhi = (jnp.dot(hlo, klh_ref[...], preferred_element_type=jnp.float32,
                   precision=_HP)
           + jnp.dot(hhi, khh_ref[...], preferred_element_type=jnp.float32,
                     precision=_HP))
    zlo_ref[...] = zlo * dinv
    zhi_ref[...] = zhi * dinv


_tc_mid = pl.pallas_call(
    _tc_mid_body,
    grid=(_GPAD,),
    in_specs=[
        pl.BlockSpec((_BF, 128), lambda i: (i, 0)),
        pl.BlockSpec((_BF, 128), lambda i: (i, 0)),
        pl.BlockSpec((128, 128), lambda i: (0, 0)),
        pl.BlockSpec((128, 128), lambda i: (0, 0)),
        pl.BlockSpec((128, 128), lambda i: (0, 0)),
        pl.BlockSpec((128, 128), lambda i: (0, 0)),
        pl.BlockSpec((1, 128), lambda i: (0, 0)),
        pl.BlockSpec((1, 128), lambda i: (0, 0)),
        pl.BlockSpec((_BF, 128), lambda i: (i, 0)),
    ],
    out_specs=(
        pl.BlockSpec((_BF, 128), lambda i: (i, 0)),
        pl.BlockSpec((_BF, 128), lambda i: (i, 0)),
    ),
    out_shape=(
        jax.ShapeDtypeStruct((_FR, 128), jnp.float32),
        jax.ShapeDtypeStruct((_FR, 128), jnp.float32),
    ),
)


def _tc_last_body(alo_ref, ahi_ref, blo_ref, bhi_ref, flo_ref, fhi_ref,
                  ssum_ref, fcb_ref, dinv_ref, y_ref):
    dinv = dinv_ref[...]
    hlo = jnp.maximum(alo_ref[...] * dinv + blo_ref[...], 0.0)
    hhi = jnp.maximum(ahi_ref[...] * dinv + bhi_ref[...], 0.0)
    t = hlo * flo_ref[...] + hhi * fhi_ref[...]
    y_ref[...] = jnp.dot(t, ssum_ref[...], preferred_element_type=jnp.float32,
                         precision=_HP) + fcb_ref[...]


_tc_last = pl.pallas_call(
    _tc_last_body,
    grid=(_GPAD,),
    in_specs=[
        pl.BlockSpec((_BF, 128), lambda i: (i, 0)),
        pl.BlockSpec((_BF, 128), lambda i: (i, 0)),
        pl.BlockSpec((1, 128), lambda i: (0, 0)),
        pl.BlockSpec((1, 128), lambda i: (0, 0)),
        pl.BlockSpec((1, 128), lambda i: (0, 0)),
        pl.BlockSpec((1, 128), lambda i: (0, 0)),
        pl.BlockSpec((128, 8), lambda i: (0, 0)),
        pl.BlockSpec((1, 8), lambda i: (0, 0)),
        pl.BlockSpec((_BF, 128), lambda i: (i, 0)),
    ],
    out_specs=pl.BlockSpec((_BF, 8), lambda i: (i, 0)),
    out_shape=jax.ShapeDtypeStruct((_FR, 8), jnp.float32),
)


def kernel(x, edge_index, W0, b0, W1, b1, W2, b2, W3, b3, fc_W, fc_b):
    src2 = edge_index[0].reshape(_R, _EC)
    dst2 = edge_index[1].reshape(_R, _EC)
    basis = jnp.zeros((_EC, _HH), jnp.float32).at[:, 0].set(1.0)
    zeros_tab = jnp.zeros((_NPAD, _HH), jnp.float32)

    eye8 = jnp.eye(8, dtype=jnp.float32)
    s_spread = jnp.kron(eye8, jnp.zeros((16, 16), jnp.float32)
                        .at[0, :].set(1.0))
    s_sum = jnp.kron(eye8, jnp.ones((16, 1), jnp.float32))

    def flat(t):
        return t.reshape(_FR, 128)

    def unflat(t):
        return t.reshape(_NPAD, _HH)

    d0, d1 = _deg_call(dst2, basis, zeros_tab)
    zlo_f, zhi_f, dinv_f = _tc_first(
        x.reshape(_XR, 1024),
        jnp.kron(eye8, W0[:, :_HH]), jnp.kron(eye8, W0[:, _HH:]),
        flat(d0), flat(d1), s_spread)
    for (W, b) in ((W1, b0), (W2, b1), (W3, b2)):
        alo, ahi = _agg_call(unflat(zlo_f), unflat(zhi_f), src2, dst2)
        zlo_f, zhi_f = _tc_mid(
            flat(alo), flat(ahi),
            jnp.kron(eye8, W[:_HH, :_HH]), jnp.kron(eye8, W[_HH:, :_HH]),
            jnp.kron(eye8, W[:_HH, _HH:]), jnp.kron(eye8, W[_HH:, _HH:]),
            jnp.tile(b[:_HH], 8).reshape(1, 128),
            jnp.tile(b[_HH:], 8).reshape(1, 128),
            dinv_f)
    alo, ahi = _agg_call(unflat(zlo_f), unflat(zhi_f), src2, dst2)
    y8 = _tc_last(
        flat(alo), flat(ahi),
        jnp.tile(b3[:_HH], 8).reshape(1, 128),
        jnp.tile(b3[_HH:], 8).reshape(1, 128),
        jnp.tile(fc_W[:_HH, 0], 8).reshape(1, 128),
        jnp.tile(fc_W[_HH:, 0], 8).reshape(1, 128),
        s_sum, jnp.tile(fc_b, 8).reshape(1, 8), dinv_f)
    return y8.reshape(_NPAD)[:_N]


# speedup vs baseline: 1.4090x; 1.1986x over previous
import jax
import jax.numpy as jnp
from jax import lax
from jax.experimental import pallas as pl
from jax.experimental.pallas import tpu as pltpu
from jax.experimental.pallas import tpu_sc as plsc

_N = 100000
_E = 1600000
_D_IN = 128
_H = 32
_HH = 16
_EC = 128
_NC = 2
_NS = 16
_NPT = 6256
_NPAD = _NPT * _NS
_R = _E // _EC
_NBLK = _R // 8

_mesh = plsc.VectorSubcoreMesh(core_axis_name="c", subcore_axis_name="s")
_sc_params = pltpu.CompilerParams(use_tc_tiling_on_sc=False)


def _deg_body(dst2, basis, zeros_tab, out0, out1,
              deg_sp, bbuf, dbuf, ssem):
    c = lax.axis_index("c")
    s = lax.axis_index("s")
    rows0 = pl.multiple_of(s * _NPT, 8)

    pltpu.sync_copy(zeros_tab.at[pl.ds(rows0, _NPT)],
                    deg_sp.at[pl.ds(rows0, _NPT)])
    pltpu.sync_copy(basis, bbuf)
    plsc.subcore_barrier()

    w = s * _NC + c
    start = 48 * w + jnp.minimum(w, 26)

    def do_block(r0, n):
        pltpu.sync_copy(dst2.at[pl.ds(r0, n)], dbuf.at[pl.ds(0, n)])
        adds = [pltpu.async_copy(bbuf, deg_sp.at[dbuf.at[j]], ssem, add=True)
                for j in range(n)]
        for a in adds:
            a.wait()

    @pl.loop(0, 48)
    def _blk(k):
        do_block(pl.multiple_of((start + k) * 8, 8), 8)

    @pl.when(w < 26)
    def _():
        do_block(pl.multiple_of((start + 48) * 8, 8), 8)

    @pl.when(w == 31)
    def _():
        do_block(_NBLK * 8, 4)

    plsc.subcore_barrier()

    @pl.when(c == 0)
    def _():
        pltpu.sync_copy(deg_sp.at[pl.ds(rows0, _NPT)],
                        out0.at[pl.ds(rows0, _NPT)])

    @pl.when(c == 1)
    def _():
        pltpu.sync_copy(deg_sp.at[pl.ds(rows0, _NPT)],
                        out1.at[pl.ds(rows0, _NPT)])


_deg_call = pl.kernel(
    _deg_body,
    out_type=(jax.ShapeDtypeStruct((_NPAD, _HH), jnp.float32),
              jax.ShapeDtypeStruct((_NPAD, _HH), jnp.float32)),
    mesh=_mesh,
    scratch_types=[
        pltpu.VMEM_SHARED((_NPAD, _HH), jnp.float32),
        pltpu.VMEM((_EC, _HH), jnp.float32),
        pltpu.VMEM((8, _EC), jnp.int32),
        pltpu.SemaphoreType.DMA,
    ],
    compiler_params=_sc_params,
)


def _agg_body(zlo, zhi, src2, dst2, outlo, outhi,
              agg_sp, sbA, dbA, sbB, dbB, msg, isem, gsem, gsem2, ssem):
    c = lax.axis_index("c")
    s = lax.axis_index("s")
    rows0 = pl.multiple_of(s * _NPT, 8)
    start = 97 * s + jnp.minimum(s, 10)

    def r0_of(k):
        return pl.multiple_of(jnp.minimum(start + k, _NBLK - 1) * 8, 8)

    def idx_fire(r0, sb, db):
        pltpu.async_copy(src2.at[pl.ds(r0, 8)], sb, isem)
        pltpu.async_copy(dst2.at[pl.ds(r0, 8)], db, isem)

    def idx_wait(sb, db):
        pltpu.make_async_copy(src2.at[pl.ds(0, 8)], sb, isem).wait()
        pltpu.make_async_copy(dst2.at[pl.ds(0, 8)], db, isem).wait()

    def run(z_ref, out_ref):
        def do_block8(sb, db):
            gA = [pltpu.async_copy(z_ref.at[sb.at[j]], msg.at[j], gsem)
                  for j in range(4)]
            gB = [pltpu.async_copy(z_ref.at[sb.at[j]], msg.at[j], gsem2)
                  for j in range(4, 8)]
            for g in gA:
                g.wait()
            sA = [pltpu.async_copy(msg.at[j], agg_sp.at[db.at[j]], ssem,
                                   add=True)
                  for j in range(4)]
            for g in gB:
                g.wait()
            for a in sA:
                a.wait()
            sB = [pltpu.async_copy(msg.at[j], agg_sp.at[db.at[j]], ssem,
                                   add=True)
                  for j in range(4, 8)]
            for a in sB:
                a.wait()

        def do_rows4(sb, db):
            gs = [pltpu.async_copy(z_ref.at[sb.at[j]], msg.at[j], gsem)
                  for j in range(4)]
            for g in gs:
                g.wait()
            adds = [pltpu.async_copy(msg.at[j], agg_sp.at[db.at[j]], ssem,
                                     add=True)
                    for j in range(4)]
            for a in adds:
                a.wait()

        pltpu.sync_copy(z_ref.at[pl.ds(rows0, _NPT)],
                        agg_sp.at[pl.ds(rows0, _NPT)])
        plsc.subcore_barrier()

        idx_fire(r0_of(0), sbA, dbA)

        @pl.loop(0, 48)
        def _blk2(t):
            kA = 2 * t
            idx_wait(sbA, dbA)
            idx_fire(r0_of(kA + 1), sbB, dbB)
            do_block8(sbA, dbA)
            idx_wait(sbB, dbB)
            idx_fire(r0_of(kA + 2), sbA, dbA)
            do_block8(sbB, dbB)

        idx_wait(sbA, dbA)
        idx_fire(r0_of(97), sbB, dbB)
        do_block8(sbA, dbA)
        idx_wait(sbB, dbB)

        @pl.when(s < 10)
        def _():
            do_block8(sbB, dbB)

        @pl.when(s == 15)
        def _():
            pltpu.sync_copy(src2.at[pl.ds(_NBLK * 8, 4)],
                            sbA.at[pl.ds(0, 4)])
            pltpu.sync_copy(dst2.at[pl.ds(_NBLK * 8, 4)],
                            dbA.at[pl.ds(0, 4)])
            do_rows4(sbA, dbA)

        plsc.subcore_barrier()
        pltpu.sync_copy(agg_sp.at[pl.ds(rows0, _NPT)],
                        out_ref.at[pl.ds(rows0, _NPT)])

    @pl.when(c == 0)
    def _():
        run(zlo, outlo)

    @pl.when(c == 1)
    def _():
        run(zhi, outhi)


_agg_call = pl.kernel(
    _agg_body,
    out_type=(jax.ShapeDtypeStruct((_NPAD, _HH), jnp.float32),
              jax.ShapeDtypeStruct((_NPAD, _HH), jnp.float32)),
    mesh=_mesh,
    scratch_types=[
        pltpu.VMEM_SHARED((_NPAD, _HH), jnp.float32),
        pltpu.VMEM((8, _EC), jnp.int32),
        pltpu.VMEM((8, _EC), jnp.int32),
        pltpu.VMEM((8, _EC), jnp.int32),
        pltpu.VMEM((8, _EC), jnp.int32),
        pltpu.VMEM((8, _EC, _HH), jnp.float32),
        pltpu.SemaphoreType.DMA,
        pltpu.SemaphoreType.DMA,
        pltpu.SemaphoreType.DMA,
        pltpu.SemaphoreType.DMA,
    ],
    compiler_params=_sc_params,
)


_FR = _NPAD * _HH // 128
_XR = _N * _D_IN // 1024
_BN = 2048
_BF = _BN * _HH // 128
_GPAD = (_FR + _BF - 1) // _BF
_HP = lax.Precision.HIGHEST


def _tc_first_body(x_ref, klo_ref, khi_ref, d0_ref, d1_ref, s_ref,
                   zlo_ref, zhi_ref, dinv_ref):
    dband = jnp.dot(d0_ref[...] + d1_ref[...], s_ref[...],
                    preferred_element_type=jnp.float32, precision=_HP)
    dinv = lax.rsqrt(dband + 1.0)
    xb = x_ref[...]
    zlo_ref[...] = jnp.dot(xb, klo_ref[...],
                           preferred_element_type=jnp.float32,
                           precision=_HP) * dinv
    zhi_ref[...] = jnp.dot(xb, khi_ref[...],
                           preferred_element_type=jnp.float32,
                           precision=_HP) * dinv
    dinv_ref[...] = dinv


_tc_first = pl.pallas_call(
    _tc_first_body,
    grid=(_GPAD,),
    in_specs=[
        pl.BlockSpec((_BF, 1024), lambda i: (i, 0)),
        pl.BlockSpec((1024, 128), lambda i: (0, 0)),
        pl.BlockSpec((1024, 128), lambda i: (0, 0)),
        pl.BlockSpec((_BF, 128), lambda i: (i, 0)),
        pl.BlockSpec((_BF, 128), lambda i: (i, 0)),
        pl.BlockSpec((128, 128), lambda i: (0, 0)),
    ],
    out_specs=(
        pl.BlockSpec((_BF, 128), lambda i: (i, 0)),
        pl.BlockSpec((_BF, 128), lambda i: (i, 0)),
        pl.BlockSpec((_BF, 128), lambda i: (i, 0)),
    ),
    out_shape=(
        jax.ShapeDtypeStruct((_FR, 128), jnp.float32),
        jax.ShapeDtypeStruct((_FR, 128), jnp.float32),
        jax.ShapeDtypeStruct((_FR, 128), jnp.float32),
    ),
)


def _tc_mid_body(alo_ref, ahi_ref, kll_ref, khl_ref, klh_ref, khh_ref,
                 blo_ref, bhi_ref, dinv_ref, zlo_ref, zhi_ref):
    dinv = dinv_ref[...]
    hlo = jnp.maximum(alo_ref[...] * dinv + blo_ref[...], 0.0)
    hhi = jnp.maximum(ahi_ref[...] * dinv + bhi_ref[...], 0.0)
    zlo = (jnp.dot(hlo, kll_ref[...], preferred_element_type=jnp.float32,
                   precision=_HP)
           + jnp.dot(hhi, khl_ref[...], preferred_element_type=jnp.float32,
                     precision=_HP))
    zhi = (jnp.dot(hlo, klh_ref[...], preferred_element_type=jnp.float32,
                   precision=_HP)
           + jnp.dot(hhi, khh_ref[...], preferred_element_type=jnp.float32,
                     precision=_HP))
    zlo_ref[...] = zlo * dinv
    zhi_ref[...] = zhi * dinv


_tc_mid = pl.pallas_call(
    _tc_mid_body,
    grid=(_GPAD,),
    in_specs=[
        pl.BlockSpec((_BF, 128), lambda i: (i, 0)),
        pl.BlockSpec((_BF, 128), lambda i: (i, 0)),
        pl.BlockSpec((128, 128), lambda i: (0, 0)),
        pl.BlockSpec((128, 128), lambda i: (0, 0)),
        pl.BlockSpec((128, 128), lambda i: (0, 0)),
        pl.BlockSpec((128, 128), lambda i: (0, 0)),
        pl.BlockSpec((1, 128), lambda i: (0, 0)),
        pl.BlockSpec((1, 128), lambda i: (0, 0)),
        pl.BlockSpec((_BF, 128), lambda i: (i, 0)),
    ],
    out_specs=(
        pl.BlockSpec((_BF, 128), lambda i: (i, 0)),
        pl.BlockSpec((_BF, 128), lambda i: (i, 0)),
    ),
    out_shape=(
        jax.ShapeDtypeStruct((_FR, 128), jnp.float32),
        jax.ShapeDtypeStruct((_FR, 128), jnp.float32),
    ),
)


def _tc_last_body(alo_ref, ahi_ref, blo_ref, bhi_ref, flo_ref, fhi_ref,
                  ssum_ref, fcb_ref, dinv_ref, y_ref):
    dinv = dinv_ref[...]
    hlo = jnp.maximum(alo_ref[...] * dinv + blo_ref[...], 0.0)
    hhi = jnp.maximum(ahi_ref[...] * dinv + bhi_ref[...], 0.0)
    t = hlo * flo_ref[...] + hhi * fhi_ref[...]
    y_ref[...] = jnp.dot(t, ssum_ref[...], preferred_element_type=jnp.float32,
                         precision=_HP) + fcb_ref[...]


_tc_last = pl.pallas_call(
    _tc_last_body,
    grid=(_GPAD,),
    in_specs=[
        pl.BlockSpec((_BF, 128), lambda i: (i, 0)),
        pl.BlockSpec((_BF, 128), lambda i: (i, 0)),
        pl.BlockSpec((1, 128), lambda i: (0, 0)),
        pl.BlockSpec((1, 128), lambda i: (0, 0)),
        pl.BlockSpec((1, 128), lambda i: (0, 0)),
        pl.BlockSpec((1, 128), lambda i: (0, 0)),
        pl.BlockSpec((128, 8), lambda i: (0, 0)),
        pl.BlockSpec((1, 8), lambda i: (0, 0)),
        pl.BlockSpec((_BF, 128), lambda i: (i, 0)),
    ],
    out_specs=pl.BlockSpec((_BF, 8), lambda i: (i, 0)),
    out_shape=jax.ShapeDtypeStruct((_FR, 8), jnp.float32),
)


def kernel(x, edge_index, W0, b0, W1, b1, W2, b2, W3, b3, fc_W, fc_b):
    src2 = edge_index[0].reshape(_R, _EC)
    dst2 = edge_index[1].reshape(_R, _EC)
    basis = jnp.zeros((_EC, _HH), jnp.float32).at[:, 0].set(1.0)
    zeros_tab = jnp.zeros((_NPAD, _HH), jnp.float32)

    eye8 = jnp.eye(8, dtype=jnp.float32)
    s_spread = jnp.kron(eye8, jnp.zeros((16, 16), jnp.float32)
                        .at[0, :].set(1.0))
    s_sum = jnp.kron(eye8, jnp.ones((16, 1), jnp.float32))

    def flat(t):
        return t.reshape(_FR, 128)

    def unflat(t):
        return t.reshape(_NPAD, _HH)

    d0, d1 = _deg_call(dst2, basis, zeros_tab)
    zlo_f, zhi_f, dinv_f = _tc_first(
        x.reshape(_XR, 1024),
        jnp.kron(eye8, W0[:, :_HH]), jnp.kron(eye8, W0[:, _HH:]),
        flat(d0), flat(d1), s_spread)
    for (W, b) in ((W1, b0), (W2, b1), (W3, b2)):
        alo, ahi = _agg_call(unflat(zlo_f), unflat(zhi_f), src2, dst2)
        zlo_f, zhi_f = _tc_mid(
            flat(alo), flat(ahi),
            jnp.kron(eye8, W[:_HH, :_HH]), jnp.kron(eye8, W[_HH:, :_HH]),
            jnp.kron(eye8, W[:_HH, _HH:]), jnp.kron(eye8, W[_HH:, _HH:]),
            jnp.tile(b[:_HH], 8).reshape(1, 128),
            jnp.tile(b[_HH:], 8).reshape(1, 128),
            dinv_f)
    alo, ahi = _agg_call(unflat(zlo_f), unflat(zhi_f), src2, dst2)
    y8 = _tc_last(
        flat(alo), flat(ahi),
        jnp.tile(b3[:_HH], 8).reshape(1, 128),
        jnp.tile(b3[_HH:], 8).reshape(1, 128),
        jnp.tile(fc_W[:_HH, 0], 8).reshape(1, 128),
        jnp.tile(fc_W[_HH:, 0], 8).reshape(1, 128),
        s_sum, jnp.tile(fc_b, 8).reshape(1, 8), dinv_f)
    return y8.reshape(_NPAD)[:_N]


# speedup vs baseline: 1.5949x; 1.1319x over previous
import jax
import jax.numpy as jnp
from jax import lax
from jax.experimental import pallas as pl
from jax.experimental.pallas import tpu as pltpu
from jax.experimental.pallas import tpu_sc as plsc

_N = 100000
_E = 1600000
_D_IN = 128
_H = 32
_HH = 16
_EC = 128
_NC = 2
_NS = 16
_NPT = 6256
_NPAD = _NPT * _NS
_R = _E // _EC
_NBLK = _R // 8

_mesh = plsc.VectorSubcoreMesh(core_axis_name="c", subcore_axis_name="s")
_sc_params = pltpu.CompilerParams(use_tc_tiling_on_sc=False)


def _deg_body(dst2, basis, zeros_tab, out0, out1,
              deg_sp, bbuf, dbuf, ssem):
    c = lax.axis_index("c")
    s = lax.axis_index("s")
    rows0 = pl.multiple_of(s * _NPT, 8)

    pltpu.sync_copy(zeros_tab.at[pl.ds(rows0, _NPT)],
                    deg_sp.at[pl.ds(rows0, _NPT)])
    pltpu.sync_copy(basis, bbuf)
    plsc.subcore_barrier()

    w = s * _NC + c
    start = 48 * w + jnp.minimum(w, 26)

    def do_block(r0, n):
        pltpu.sync_copy(dst2.at[pl.ds(r0, n)], dbuf.at[pl.ds(0, n)])
        adds = [pltpu.async_copy(bbuf, deg_sp.at[dbuf.at[j]], ssem, add=True)
                for j in range(n)]
        for a in adds:
            a.wait()

    @pl.loop(0, 48)
    def _blk(k):
        do_block(pl.multiple_of((start + k) * 8, 8), 8)

    @pl.when(w < 26)
    def _():
        do_block(pl.multiple_of((start + 48) * 8, 8), 8)

    @pl.when(w == 31)
    def _():
        do_block(_NBLK * 8, 4)

    plsc.subcore_barrier()

    @pl.when(c == 0)
    def _():
        pltpu.sync_copy(deg_sp.at[pl.ds(rows0, _NPT)],
                        out0.at[pl.ds(rows0, _NPT)])

    @pl.when(c == 1)
    def _():
        pltpu.sync_copy(deg_sp.at[pl.ds(rows0, _NPT)],
                        out1.at[pl.ds(rows0, _NPT)])


_deg_call = pl.kernel(
    _deg_body,
    out_type=(jax.ShapeDtypeStruct((_NPAD, _HH), jnp.float32),
              jax.ShapeDtypeStruct((_NPAD, _HH), jnp.float32)),
    mesh=_mesh,
    scratch_types=[
        pltpu.VMEM_SHARED((_NPAD, _HH), jnp.float32),
        pltpu.VMEM((_EC, _HH), jnp.float32),
        pltpu.VMEM((8, _EC), jnp.int32),
        pltpu.SemaphoreType.DMA,
    ],
    compiler_params=_sc_params,
)


def _agg_body(zlo, zhi, src2, dst2, outlo, outhi,
              agg_sp, sbA, dbA, sbB, dbB, msg, isem, gsem, gsem2, ssem):
    c = lax.axis_index("c")
    s = lax.axis_index("s")
    rows0 = pl.multiple_of(s * _NPT, 8)
    start = 97 * s + jnp.minimum(s, 10)

    def r0_of(k):
        return pl.multiple_of(jnp.minimum(start + k, _NBLK - 1) * 8, 8)

    def idx_fire(r0, sb, db):
        pltpu.async_copy(src2.at[pl.ds(r0, 8)], sb, isem)
        pltpu.async_copy(dst2.at[pl.ds(r0, 8)], db, isem)

    def idx_wait(sb, db):
        pltpu.make_async_copy(src2.at[pl.ds(0, 8)], sb, isem).wait()
        pltpu.make_async_copy(dst2.at[pl.ds(0, 8)], db, isem).wait()

    def run(z_ref, out_ref):
        def do_block8_head(sb, db):
            gA = [pltpu.async_copy(z_ref.at[sb.at[j]], msg.at[j], gsem)
                  for j in range(4)]
            gB = [pltpu.async_copy(z_ref.at[sb.at[j]], msg.at[j], gsem2)
                  for j in range(4, 8)]
            for g in gA:
                g.wait()
            sA = [pltpu.async_copy(msg.at[j], agg_sp.at[db.at[j]], ssem,
                                   add=True)
                  for j in range(4)]
            for g in gB:
                g.wait()
            for a in sA:
                a.wait()
            for j in range(4, 8):
                pltpu.async_copy(msg.at[j], agg_sp.at[db.at[j]], ssem,
                                 add=True)

        def do_block8(sb, db, prefetch):
            gA = [pltpu.async_copy(z_ref.at[sb.at[j]], msg.at[j], gsem)
                  for j in range(4)]
            for j in range(4, 8):
                pltpu.make_async_copy(msg.at[j], agg_sp.at[db.at[j]],
                                      ssem).wait()
            prefetch()
            gB = [pltpu.async_copy(z_ref.at[sb.at[j]], msg.at[j], gsem2)
                  for j in range(4, 8)]
            for g in gA:
                g.wait()
            sA = [pltpu.async_copy(msg.at[j], agg_sp.at[db.at[j]], ssem,
                                   add=True)
                  for j in range(4)]
            for g in gB:
                g.wait()
            for a in sA:
                a.wait()
            for j in range(4, 8):
                pltpu.async_copy(msg.at[j], agg_sp.at[db.at[j]], ssem,
                                 add=True)

        def drain_sB(db):
            for j in range(4, 8):
                pltpu.make_async_copy(msg.at[j], agg_sp.at[db.at[j]],
                                      ssem).wait()

        def do_rows4(sb, db):
            gs = [pltpu.async_copy(z_ref.at[sb.at[j]], msg.at[j], gsem)
                  for j in range(4)]
            for g in gs:
                g.wait()
            adds = [pltpu.async_copy(msg.at[j], agg_sp.at[db.at[j]], ssem,
                                     add=True)
                    for j in range(4)]
            for a in adds:
                a.wait()

        pltpu.sync_copy(z_ref.at[pl.ds(rows0, _NPT)],
                        agg_sp.at[pl.ds(rows0, _NPT)])
        plsc.subcore_barrier()

        idx_fire(r0_of(0), sbA, dbA)
        idx_wait(sbA, dbA)
        idx_fire(r0_of(1), sbB, dbB)
        do_block8_head(sbA, dbA)

        @pl.loop(0, 48)
        def _blk2(t):
            idx_wait(sbB, dbB)
            do_block8(sbB, dbB,
                      lambda: idx_fire(r0_of(2 * t + 2), sbA, dbA))
            idx_wait(sbA, dbA)
            do_block8(sbA, dbA,
                      lambda: idx_fire(r0_of(2 * t + 3), sbB, dbB))

        idx_wait(sbB, dbB)

        @pl.when(s < 10)
        def _():
            do_block8(sbB, dbB, lambda: None)

        drain_sB(dbB)

        @pl.when(s == 15)
        def _():
            pltpu.sync_copy(src2.at[pl.ds(_NBLK * 8, 4)],
                            sbA.at[pl.ds(0, 4)])
            pltpu.sync_copy(dst2.at[pl.ds(_NBLK * 8, 4)],
                            dbA.at[pl.ds(0, 4)])
            do_rows4(sbA, dbA)

        plsc.subcore_barrier()
        pltpu.sync_copy(agg_sp.at[pl.ds(rows0, _NPT)],
                        out_ref.at[pl.ds(rows0, _NPT)])

    @pl.when(c == 0)
    def _():
        run(zlo, outlo)

    @pl.when(c == 1)
    def _():
        run(zhi, outhi)


_agg_call = pl.kernel(
    _agg_body,
    out_type=(jax.ShapeDtypeStruct((_NPAD, _HH), jnp.float32),
              jax.ShapeDtypeStruct((_NPAD, _HH), jnp.float32)),
    mesh=_mesh,
    scratch_types=[
        pltpu.VMEM_SHARED((_NPAD, _HH), jnp.float32),
        pltpu.VMEM((8, _EC), jnp.int32),
        pltpu.VMEM((8, _EC), jnp.int32),
        pltpu.VMEM((8, _EC), jnp.int32),
        pltpu.VMEM((8, _EC), jnp.int32),
        pltpu.VMEM((8, _EC, _HH), jnp.float32),
        pltpu.SemaphoreType.DMA,
        pltpu.SemaphoreType.DMA,
        pltpu.SemaphoreType.DMA,
        pltpu.SemaphoreType.DMA,
    ],
    compiler_params=_sc_params,
)


_FR = _NPAD * _HH // 128
_XR = _N * _D_IN // 1024
_BN = 2048
_BF = _BN * _HH // 128
_GPAD = (_FR + _BF - 1) // _BF
_HP = lax.Precision.HIGHEST


def _tc_first_body(x_ref, klo_ref, khi_ref, d0_ref, d1_ref, s_ref,
                   zlo_ref, zhi_ref, dinv_ref):
    dband = jnp.dot(d0_ref[...] + d1_ref[...], s_ref[...],
                    preferred_element_type=jnp.float32, precision=_HP)
    dinv = lax.rsqrt(dband + 1.0)
    xb = x_ref[...]
    zlo_ref[...] = jnp.dot(xb, klo_ref[...],
                           preferred_element_type=jnp.float32,
                           precision=_HP) * dinv
    zhi_ref[...] = jnp.dot(xb, khi_ref[...],
                           preferred_element_type=jnp.float32,
                           precision=_HP) * dinv
    dinv_ref[...] = dinv


_tc_first = pl.pallas_call(
    _tc_first_body,
    grid=(_GPAD,),
    in_specs=[
        pl.BlockSpec((_BF, 1024), lambda i: (i, 0)),
        pl.BlockSpec((1024, 128), lambda i: (0, 0)),
        pl.BlockSpec((1024, 128), lambda i: (0, 0)),
        pl.BlockSpec((_BF, 128), lambda i: (i, 0)),
        pl.BlockSpec((_BF, 128), lambda i: (i, 0)),
        pl.BlockSpec((128, 128), lambda i: (0, 0)),
    ],
    out_specs=(
        pl.BlockSpec((_BF, 128), lambda i: (i, 0)),
        pl.BlockSpec((_BF, 128), lambda i: (i, 0)),
        pl.BlockSpec((_BF, 128), lambda i: (i, 0)),
    ),
    out_shape=(
        jax.ShapeDtypeStruct((_FR, 128), jnp.float32),
        jax.ShapeDtypeStruct((_FR, 128), jnp.float32),
        jax.ShapeDtypeStruct((_FR, 128), jnp.float32),
    ),
)


def _tc_mid_body(alo_ref, ahi_ref, kll_ref, khl_ref, klh_ref, khh_ref,
                 blo_ref, bhi_ref, dinv_ref, zlo_ref, zhi_ref):
    dinv = dinv_ref[...]
    hlo = jnp.maximum(alo_ref[...] * dinv + blo_ref[...], 0.0)
    hhi = jnp.maximum(ahi_ref[...] * dinv + bhi_ref[...], 0.0)
    zlo = (jnp.dot(hlo, kll_ref[...], preferred_element_type=jnp.float32,
                   precision=_HP)
           + jnp.dot(hhi, khl_ref[...], preferred_element_type=jnp.float32,
                     precision=_HP))
    zhi = (jnp.dot(hlo, klh_ref[...], preferred_element_type=jnp.float32,
                   precision=_HP)
           + jnp.dot(hhi, khh_ref[...], preferred_element_type=jnp.float32,
                     precision=_HP))
    zlo_ref[...] = zlo * dinv
    zhi_ref[...] = zhi * dinv


_tc_mid = pl.pallas_call(
    _tc_mid_body,
    grid=(_GPAD,),
    in_specs=[
        pl.BlockSpec((_BF, 128), lambda i: (i, 0)),
        pl.BlockSpec((_BF, 128), lambda i: (i, 0)),
        pl.BlockSpec((128, 128), lambda i: (0, 0)),
        pl.BlockSpec((128, 128), lambda i: (0, 0)),
        pl.BlockSpec((128, 128), lambda i: (0, 0)),
        pl.BlockSpec((128, 128), lambda i: (0, 0)),
        pl.BlockSpec((1, 128), lambda i: (0, 0)),
        pl.BlockSpec((1, 128), lambda i: (0, 0)),
        pl.BlockSpec((_BF, 128), lambda i: (i, 0)),
    ],
    out_specs=(
        pl.BlockSpec((_BF, 128), lambda i: (i, 0)),
        pl.BlockSpec((_BF, 128), lambda i: (i, 0)),
    ),
    out_shape=(
        jax.ShapeDtypeStruct((_FR, 128), jnp.float32),
        jax.ShapeDtypeStruct((_FR, 128), jnp.float32),
    ),
)


def _tc_last_body(alo_ref, ahi_ref, blo_ref, bhi_ref, flo_ref, fhi_ref,
                  ssum_ref, fcb_ref, dinv_ref, y_ref):
    dinv = dinv_ref[...]
    hlo = jnp.maximum(alo_ref[...] * dinv + blo_ref[...], 0.0)
    hhi = jnp.maximum(ahi_ref[...] * dinv + bhi_ref[...], 0.0)
    t = hlo * flo_ref[...] + hhi * fhi_ref[...]
    y_ref[...] = jnp.dot(t, ssum_ref[...], preferred_element_type=jnp.float32,
                         precision=_HP) + fcb_ref[...]


_tc_last = pl.pallas_call(
    _tc_last_body,
    grid=(_GPAD,),
    in_specs=[
        pl.BlockSpec((_BF, 128), lambda i: (i, 0)),
        pl.BlockSpec((_BF, 128), lambda i: (i, 0)),
        pl.BlockSpec((1, 128), lambda i: (0, 0)),
        pl.BlockSpec((1, 128), lambda i: (0, 0)),
        pl.BlockSpec((1, 128), lambda i: (0, 0)),
        pl.BlockSpec((1, 128), lambda i: (0, 0)),
        pl.BlockSpec((128, 8), lambda i: (0, 0)),
        pl.BlockSpec((1, 8), lambda i: (0, 0)),
        pl.BlockSpec((_BF, 128), lambda i: (i, 0)),
    ],
    out_specs=pl.BlockSpec((_BF, 8), lambda i: (i, 0)),
    out_shape=jax.ShapeDtypeStruct((_FR, 8), jnp.float32),
)


def kernel(x, edge_index, W0, b0, W1, b1, W2, b2, W3, b3, fc_W, fc_b):
    src2 = edge_index[0].reshape(_R, _EC)
    dst2 = edge_index[1].reshape(_R, _EC)
    basis = jnp.zeros((_EC, _HH), jnp.float32).at[:, 0].set(1.0)
    zeros_tab = jnp.zeros((_NPAD, _HH), jnp.float32)

    eye8 = jnp.eye(8, dtype=jnp.float32)
    s_spread = jnp.kron(eye8, jnp.zeros((16, 16), jnp.float32)
                        .at[0, :].set(1.0))
    s_sum = jnp.kron(eye8, jnp.ones((16, 1), jnp.float32))

    def flat(t):
        return t.reshape(_FR, 128)

    def unflat(t):
        return t.reshape(_NPAD, _HH)

    d0, d1 = _deg_call(dst2, basis, zeros_tab)
    zlo_f, zhi_f, dinv_f = _tc_first(
        x.reshape(_XR, 1024),
        jnp.kron(eye8, W0[:, :_HH]), jnp.kron(eye8, W0[:, _HH:]),
        flat(d0), flat(d1), s_spread)
    for (W, b) in ((W1, b0), (W2, b1), (W3, b2)):
        alo, ahi = _agg_call(unflat(zlo_f), unflat(zhi_f), src2, dst2)
        zlo_f, zhi_f = _tc_mid(
            flat(alo), flat(ahi),
            jnp.kron(eye8, W[:_HH, :_HH]), jnp.kron(eye8, W[_HH:, :_HH]),
            jnp.kron(eye8, W[:_HH, _HH:]), jnp.kron(eye8, W[_HH:, _HH:]),
            jnp.tile(b[:_HH], 8).reshape(1, 128),
            jnp.tile(b[_HH:], 8).reshape(1, 128),
            dinv_f)
    alo, ahi = _agg_call(unflat(zlo_f), unflat(zhi_f), src2, dst2)
    y8 = _tc_last(
        flat(alo), flat(ahi),
        jnp.tile(b3[:_HH], 8).reshape(1, 128),
        jnp.tile(b3[_HH:], 8).reshape(1, 128),
        jnp.tile(fc_W[:_HH, 0], 8).reshape(1, 128),
        jnp.tile(fc_W[_HH:, 0], 8).reshape(1, 128),
        s_sum, jnp.tile(fc_b, 8).reshape(1, 8), dinv_f)
    return y8.reshape(_NPAD)[:_N]


# speedup vs baseline: 1.6284x; 1.0211x over previous
import jax
import jax.numpy as jnp
from jax import lax
from jax.experimental import pallas as pl
from jax.experimental.pallas import tpu as pltpu
from jax.experimental.pallas import tpu_sc as plsc

_N = 100000
_E = 1600000
_D_IN = 128
_H = 32
_HH = 16
_EC = 128
_NC = 2
_NS = 16
_NPT = 6256
_NPAD = _NPT * _NS
_R = _E // _EC
_NBLK = _R // 8

_mesh = plsc.VectorSubcoreMesh(core_axis_name="c", subcore_axis_name="s")
_sc_params = pltpu.CompilerParams(use_tc_tiling_on_sc=False)


def _deg_body(dst2, basis, zeros_tab, out0, out1,
              deg_sp, bbuf, dbA, dbB, isem, ssem):
    c = lax.axis_index("c")
    s = lax.axis_index("s")
    rows0 = pl.multiple_of(s * _NPT, 8)

    pltpu.sync_copy(zeros_tab.at[pl.ds(rows0, _NPT)],
                    deg_sp.at[pl.ds(rows0, _NPT)])
    pltpu.sync_copy(basis, bbuf)
    plsc.subcore_barrier()

    w = s * _NC + c
    start = 48 * w + jnp.minimum(w, 26)

    def r0c(k):
        return pl.multiple_of(jnp.minimum(start + k, _NBLK - 1) * 8, 8)

    def ifire(r0, db):
        pltpu.async_copy(dst2.at[pl.ds(r0, 8)], db, isem)

    def iwait(db):
        pltpu.make_async_copy(dst2.at[pl.ds(0, 8)], db, isem).wait()

    def blk(db, n):
        adds = [pltpu.async_copy(bbuf, deg_sp.at[db.at[j]], ssem, add=True)
                for j in range(n)]
        for a in adds:
            a.wait()

    ifire(r0c(0), dbA)

    @pl.loop(0, 24)
    def _blk2(t):
        iwait(dbA)
        ifire(r0c(2 * t + 1), dbB)
        blk(dbA, 8)
        iwait(dbB)
        ifire(r0c(2 * t + 2), dbA)
        blk(dbB, 8)

    iwait(dbA)

    @pl.when(w < 26)
    def _():
        blk(dbA, 8)

    @pl.when(w == 31)
    def _():
        pltpu.sync_copy(dst2.at[pl.ds(_NBLK * 8, 4)], dbB.at[pl.ds(0, 4)])
        blk(dbB, 4)

    plsc.subcore_barrier()

    @pl.when(c == 0)
    def _():
        pltpu.sync_copy(deg_sp.at[pl.ds(rows0, _NPT)],
                        out0.at[pl.ds(rows0, _NPT)])

    @pl.when(c == 1)
    def _():
        pltpu.sync_copy(deg_sp.at[pl.ds(rows0, _NPT)],
                        out1.at[pl.ds(rows0, _NPT)])


_deg_call = pl.kernel(
    _deg_body,
    out_type=(jax.ShapeDtypeStruct((_NPAD, _HH), jnp.float32),
              jax.ShapeDtypeStruct((_NPAD, _HH), jnp.float32)),
    mesh=_mesh,
    scratch_types=[
        pltpu.VMEM_SHARED((_NPAD, _HH), jnp.float32),
        pltpu.VMEM((_EC, _HH), jnp.float32),
        pltpu.VMEM((8, _EC), jnp.int32),
        pltpu.VMEM((8, _EC), jnp.int32),
        pltpu.SemaphoreType.DMA,
        pltpu.SemaphoreType.DMA,
    ],
    compiler_params=_sc_params,
)


def _agg_body(zlo, zhi, src2, dst2, outlo, outhi,
              agg_sp, sbA, dbA, sbB, dbB, msg, isem, gsem, gsem2, ssem):
    c = lax.axis_index("c")
    s = lax.axis_index("s")
    rows0 = pl.multiple_of(s * _NPT, 8)
    start = 97 * s + jnp.minimum(s, 10)

    def r0_of(k):
        return pl.multiple_of(jnp.minimum(start + k, _NBLK - 1) * 8, 8)

    def idx_fire(r0, sb, db):
        pltpu.async_copy(src2.at[pl.ds(r0, 8)], sb, isem)
        pltpu.async_copy(dst2.at[pl.ds(r0, 8)], db, isem)

    def idx_wait(sb, db):
        pltpu.make_async_copy(src2.at[pl.ds(0, 8)], sb, isem).wait()
        pltpu.make_async_copy(dst2.at[pl.ds(0, 8)], db, isem).wait()

    def run(z_ref, out_ref):
        def do_block8_head(sb, db):
            gA = [pltpu.async_copy(z_ref.at[sb.at[j]], msg.at[j], gsem)
                  for j in range(4)]
            gB = [pltpu.async_copy(z_ref.at[sb.at[j]], msg.at[j], gsem2)
                  for j in range(4, 8)]
            for g in gA:
                g.wait()
            sA = [pltpu.async_copy(msg.at[j], agg_sp.at[db.at[j]], ssem,
                                   add=True)
                  for j in range(4)]
            for g in gB:
                g.wait()
            for a in sA:
                a.wait()
            for j in range(4, 8):
                pltpu.async_copy(msg.at[j], agg_sp.at[db.at[j]], ssem,
                                 add=True)

        def do_block8(sb, db, prefetch):
            gA = [pltpu.async_copy(z_ref.at[sb.at[j]], msg.at[j], gsem)
                  for j in range(4)]
            for j in range(4, 8):
                pltpu.make_async_copy(msg.at[j], agg_sp.at[db.at[j]],
                                      ssem).wait()
            prefetch()
            gB = [pltpu.async_copy(z_ref.at[sb.at[j]], msg.at[j], gsem2)
                  for j in range(4, 8)]
            for g in gA:
                g.wait()
            sA = [pltpu.async_copy(msg.at[j], agg_sp.at[db.at[j]], ssem,
                                   add=True)
                  for j in range(4)]
            for g in gB:
                g.wait()
            for a in sA:
                a.wait()
            for j in range(4, 8):
                pltpu.async_copy(msg.at[j], agg_sp.at[db.at[j]], ssem,
                                 add=True)

        def drain_sB(db):
            for j in range(4, 8):
                pltpu.make_async_copy(msg.at[j], agg_sp.at[db.at[j]],
                                      ssem).wait()

        def do_rows4(sb, db):
            gs = [pltpu.async_copy(z_ref.at[sb.at[j]], msg.at[j], gsem)
                  for j in range(4)]
            for g in gs:
                g.wait()
            adds = [pltpu.async_copy(msg.at[j], agg_sp.at[db.at[j]], ssem,
                                     add=True)
                    for j in range(4)]
            for a in adds:
                a.wait()

        pltpu.sync_copy(z_ref.at[pl.ds(rows0, _NPT)],
                        agg_sp.at[pl.ds(rows0, _NPT)])
        plsc.subcore_barrier()

        idx_fire(r0_of(0), sbA, dbA)
        idx_wait(sbA, dbA)
        idx_fire(r0_of(1), sbB, dbB)
        do_block8_head(sbA, dbA)

        @pl.loop(0, 48)
        def _blk2(t):
            idx_wait(sbB, dbB)
            do_block8(sbB, dbB,
                      lambda: idx_fire(r0_of(2 * t + 2), sbA, dbA))
            idx_wait(sbA, dbA)
            do_block8(sbA, dbA,
                      lambda: idx_fire(r0_of(2 * t + 3), sbB, dbB))

        idx_wait(sbB, dbB)

        @pl.when(s < 10)
        def _():
            do_block8(sbB, dbB, lambda: None)

        drain_sB(dbB)

        @pl.when(s == 15)
        def _():
            pltpu.sync_copy(src2.at[pl.ds(_NBLK * 8, 4)],
                            sbA.at[pl.ds(0, 4)])
            pltpu.sync_copy(dst2.at[pl.ds(_NBLK * 8, 4)],
                            dbA.at[pl.ds(0, 4)])
            do_rows4(sbA, dbA)

        plsc.subcore_barrier()
        pltpu.sync_copy(agg_sp.at[pl.ds(rows0, _NPT)],
                        out_ref.at[pl.ds(rows0, _NPT)])

    @pl.when(c == 0)
    def _():
        run(zlo, outlo)

    @pl.when(c == 1)
    def _():
        run(zhi, outhi)


_agg_call = pl.kernel(
    _agg_body,
    out_type=(jax.ShapeDtypeStruct((_NPAD, _HH), jnp.float32),
              jax.ShapeDtypeStruct((_NPAD, _HH), jnp.float32)),
    mesh=_mesh,
    scratch_types=[
        pltpu.VMEM_SHARED((_NPAD, _HH), jnp.float32),
        pltpu.VMEM((8, _EC), jnp.int32),
        pltpu.VMEM((8, _EC), jnp.int32),
        pltpu.VMEM((8, _EC), jnp.int32),
        pltpu.VMEM((8, _EC), jnp.int32),
        pltpu.VMEM((8, _EC, _HH), jnp.float32),
        pltpu.SemaphoreType.DMA,
        pltpu.SemaphoreType.DMA,
        pltpu.SemaphoreType.DMA,
        pltpu.SemaphoreType.DMA,
    ],
    compiler_params=_sc_params,
)


_FR = _NPAD * _HH // 128
_XR = _N * _D_IN // 1024
_BN = 2048
_BF = _BN * _HH // 128
_GPAD = (_FR + _BF - 1) // _BF
_HP = lax.Precision.HIGHEST


def _tc_first_body(x_ref, klo_ref, khi_ref, d0_ref, d1_ref, s_ref,
                   zlo_ref, zhi_ref, dinv_ref):
    dband = jnp.dot(d0_ref[...] + d1_ref[...], s_ref[...],
                    preferred_element_type=jnp.float32, precision=_HP)
    dinv = lax.rsqrt(dband + 1.0)
    xb = x_ref[...]
    zlo_ref[...] = jnp.dot(xb, klo_ref[...],
                           preferred_element_type=jnp.float32,
                           precision=_HP) * dinv
    zhi_ref[...] = jnp.dot(xb, khi_ref[...],
                           preferred_element_type=jnp.float32,
                           precision=_HP) * dinv
    dinv_ref[...] = dinv


_tc_first = pl.pallas_call(
    _tc_first_body,
    grid=(_GPAD,),
    in_specs=[
        pl.BlockSpec((_BF, 1024), lambda i: (i, 0)),
        pl.BlockSpec((1024, 128), lambda i: (0, 0)),
        pl.BlockSpec((1024, 128), lambda i: (0, 0)),
        pl.BlockSpec((_BF, 128), lambda i: (i, 0)),
        pl.BlockSpec((_BF, 128), lambda i: (i, 0)),
        pl.BlockSpec((128, 128), lambda i: (0, 0)),
    ],
    out_specs=(
        pl.BlockSpec((_BF, 128), lambda i: (i, 0)),
        pl.BlockSpec((_BF, 128), lambda i: (i, 0)),
        pl.BlockSpec((_BF, 128), lambda i: (i, 0)),
    ),
    out_shape=(
        jax.ShapeDtypeStruct((_FR, 128), jnp.float32),
        jax.ShapeDtypeStruct((_FR, 128), jnp.float32),
        jax.ShapeDtypeStruct((_FR, 128), jnp.float32),
    ),
)


def _tc_mid_body(alo_ref, ahi_ref, kll_ref, khl_ref, klh_ref, khh_ref,
                 blo_ref, bhi_ref, dinv_ref, zlo_ref, zhi_ref):
    dinv = dinv_ref[...]
    hlo = jnp.maximum(alo_ref[...] * dinv + blo_ref[...], 0.0)
    hhi = jnp.maximum(ahi_ref[...] * dinv + bhi_ref[...], 0.0)
    zlo = (jnp.dot(hlo, kll_ref[...], preferred_element_type=jnp.float32,
                   precision=_HP)
           + jnp.dot(hhi, khl_ref[...], preferred_element_type=jnp.float32,
                     precision=_HP))
    zhi = (jnp.dot(hlo, klh_ref[...], preferred_element_type=jnp.float32,
                   precision=_HP)
           + jnp.dot(hhi, khh_ref[...], preferred_element_type=jnp.float32,
                     precision=_HP))
    zlo_ref[...] = zlo * dinv
    zhi_ref[...] = zhi * dinv


_tc_mid = pl.pallas_call(
    _tc_mid_body,
    grid=(_GPAD,),
    in_specs=[
        pl.BlockSpec((_BF, 128), lambda i: (i, 0)),
        pl.BlockSpec((_BF, 128), lambda i: (i, 0)),
        pl.BlockSpec((128, 128), lambda i: (0, 0)),
        pl.BlockSpec((128, 128), lambda i: (0, 0)),
        pl.BlockSpec((128, 128), lambda i: (0, 0)),
        pl.BlockSpec((128, 128), lambda i: (0, 0)),
        pl.BlockSpec((1, 128), lambda i: (0, 0)),
        pl.BlockSpec((1, 128), lambda i: (0, 0)),
        pl.BlockSpec((_BF, 128), lambda i: (i, 0)),
    ],
    out_specs=(
        pl.BlockSpec((_BF, 128), lambda i: (i, 0)),
        pl.BlockSpec((_BF, 128), lambda i: (i, 0)),
    ),
    out_shape=(
        jax.ShapeDtypeStruct((_FR, 128), jnp.float32),
        jax.ShapeDtypeStruct((_FR, 128), jnp.float32),
    ),
)


def _tc_last_body(alo_ref, ahi_ref, blo_ref, bhi_ref, flo_ref, fhi_ref,
                  ssum_ref, fcb_ref, dinv_ref, y_ref):
    dinv = dinv_ref[...]
    hlo = jnp.maximum(alo_ref[...] * dinv + blo_ref[...], 0.0)
    hhi = jnp.maximum(ahi_ref[...] * dinv + bhi_ref[...], 0.0)
    t = hlo * flo_ref[...] + hhi * fhi_ref[...]
    y_ref[...] = jnp.dot(t, ssum_ref[...], preferred_element_type=jnp.float32,
                         precision=_HP) + fcb_ref[...]


_tc_last = pl.pallas_call(
    _tc_last_body,
    grid=(_GPAD,),
    in_specs=[
        pl.BlockSpec((_BF, 128), lambda i: (i, 0)),
        pl.BlockSpec((_BF, 128), lambda i: (i, 0)),
        pl.BlockSpec((1, 128), lambda i: (0, 0)),
        pl.BlockSpec((1, 128), lambda i: (0, 0)),
        pl.BlockSpec((1, 128), lambda i: (0, 0)),
        pl.BlockSpec((1, 128), lambda i: (0, 0)),
        pl.BlockSpec((128, 8), lambda i: (0, 0)),
        pl.BlockSpec((1, 8), lambda i: (0, 0)),
        pl.BlockSpec((_BF, 128), lambda i: (i, 0)),
    ],
    out_specs=pl.BlockSpec((_BF, 8), lambda i: (i, 0)),
    out_shape=jax.ShapeDtypeStruct((_FR, 8), jnp.float32),
)


def kernel(x, edge_index, W0, b0, W1, b1, W2, b2, W3, b3, fc_W, fc_b):
    src2 = edge_index[0].reshape(_R, _EC)
    dst2 = edge_index[1].reshape(_R, _EC)
    basis = jnp.zeros((_EC, _HH), jnp.float32).at[:, 0].set(1.0)
    zeros_tab = jnp.zeros((_NPAD, _HH), jnp.float32)

    eye8 = jnp.eye(8, dtype=jnp.float32)
    s_spread = jnp.kron(eye8, jnp.zeros((16, 16), jnp.float32)
                        .at[0, :].set(1.0))
    s_sum = jnp.kron(eye8, jnp.ones((16, 1), jnp.float32))

    def flat(t):
        return t.reshape(_FR, 128)

    def unflat(t):
        return t.reshape(_NPAD, _HH)

    d0, d1 = _deg_call(dst2, basis, zeros_tab)
    zlo_f, zhi_f, dinv_f = _tc_first(
        x.reshape(_XR, 1024),
        jnp.kron(eye8, W0[:, :_HH]), jnp.kron(eye8, W0[:, _HH:]),
        flat(d0), flat(d1), s_spread)
    for (W, b) in ((W1, b0), (W2, b1), (W3, b2)):
        alo, ahi = _agg_call(unflat(zlo_f), unflat(zhi_f), src2, dst2)
        zlo_f, zhi_f = _tc_mid(
            flat(alo), flat(ahi),
            jnp.kron(eye8, W[:_HH, :_HH]), jnp.kron(eye8, W[_HH:, :_HH]),
            jnp.kron(eye8, W[:_HH, _HH:]), jnp.kron(eye8, W[_HH:, _HH:]),
            jnp.tile(b[:_HH], 8).reshape(1, 128),
            jnp.tile(b[_HH:], 8).reshape(1, 128),
            dinv_f)
    alo, ahi = _agg_call(unflat(zlo_f), unflat(zhi_f), src2, dst2)
    y8 = _tc_last(
        flat(alo), flat(ahi),
        jnp.tile(b3[:_HH], 8).reshape(1, 128),
        jnp.tile(b3[_HH:], 8).reshape(1, 128),
        jnp.tile(fc_W[:_HH, 0], 8).reshape(1, 128),
        jnp.tile(fc_W[_HH:, 0], 8).reshape(1, 128),
        s_sum, jnp.tile(fc_b, 8).reshape(1, 8), dinv_f)
    return y8.reshape(_NPAD)[:_N]
